# baseline (device time: 54176 ns/iter reference)
import jax
import jax.numpy as jnp
from jax import lax
from jax.experimental import pallas as pl
from jax.experimental.pallas import tpu as pltpu

N_DEV = 16
M = 1024
N = 1024
Q = M // 4
S = M // 16
R2 = Q // 2


def kernel(x, w_mat):
    def body(x_ref, w_ref, out_ref, s1_buf, e1_buf, e2_buf, s4_buf,
             s1r_ss, s1r_rs, s1l_ss, s1l_rs,
             ex_ss, ex_rs,
             s4r_ss, s4r_rs, s4l_ss, s4l_rs):
        my = lax.axis_index("i")
        z = my // 4
        p = lax.rem(my, 4)
        zb0 = lax.rem(z, 2)
        zb1 = z // 2
        p_r = z * 4 + lax.rem(p + 1, 4)
        p_l = z * 4 + lax.rem(p + 3, 4)
        d1 = (z ^ 1) * 4 + p
        d2 = (z ^ 2) * 4 + p

        bs = pltpu.get_barrier_semaphore()
        for nbr in (p_l, p_r, d1, d2):
            pl.semaphore_signal(
                bs, inc=1, device_id=(nbr,),
                device_id_type=pl.DeviceIdType.MESH,
            )
        pl.semaphore_wait(bs, 4)

        pending = []

        def start(src, dst, ssem, rsem, dev):
            r = pltpu.make_async_remote_copy(
                src_ref=src, dst_ref=dst, send_sem=ssem, recv_sem=rsem,
                device_id=(dev,), device_id_type=pl.DeviceIdType.MESH,
            )
            r.start()
            pending.append(r)
            return r

        wv = w_ref[...]

        def gemm(row0, nrows):
            return jnp.dot(
                x_ref[pl.ds(row0, nrows), :], wv,
                preferred_element_type=jnp.float32,
            )

        s1_buf[0, :, :] = gemm(p * Q, Q).astype(jnp.bfloat16)
        for s in range(3):
            rr = start(s1_buf.at[s, pl.ds(0, R2), :],
                       s1_buf.at[s + 1, pl.ds(0, R2), :],
                       s1r_ss.at[s], s1r_rs.at[s], p_r)
            rl = start(s1_buf.at[s, pl.ds(R2, R2), :],
                       s1_buf.at[s + 1, pl.ds(R2, R2), :],
                       s1l_ss.at[s], s1l_rs.at[s], p_l)
            iR = lax.rem(p + 3 - s, 4)
            iL = lax.rem(p + 1 + s, 4)
            contrib = jnp.concatenate(
                [gemm(iR * Q, R2), gemm(iL * Q + R2, R2)], axis=0
            )
            rr.wait_recv()
            rl.wait_recv()
            s1_buf[s + 1, :, :] = (
                s1_buf[s + 1, :, :].astype(jnp.float32) + contrib
            ).astype(jnp.bfloat16)
        qR = lax.rem(p + 1, 4)
        qL = lax.rem(p + 3, 4)

        def out_base(j):
            return jnp.where(j < 2, qR, qL) * Q + j * S

        kh = zb0 * R2
        oh = (1 - zb0) * R2
        ex1 = start(s1_buf.at[3, pl.ds(oh, R2), :], e1_buf,
                    ex_ss.at[0], ex_rs.at[0], d1)
        ex1.wait_recv()
        s1_buf[3, pl.ds(kh, R2), :] = (
            s1_buf[3, pl.ds(kh, R2), :].astype(jnp.float32)
            + e1_buf[...].astype(jnp.float32)
        ).astype(jnp.bfloat16)

        kb = kh + zb1 * S
        sb = kh + (1 - zb1) * S
        ex2 = start(s1_buf.at[3, pl.ds(sb, S), :], e2_buf,
                    ex_ss.at[1], ex_rs.at[1], d2)
        ex2.wait_recv()
        red = (s1_buf[3, pl.ds(kb, S), :].astype(jnp.float32)
               + e2_buf[...].astype(jnp.float32))
        silu = red * jax.nn.sigmoid(red)
        own16 = zb0 * 2 + zb1
        p16 = zb0 * 2 + (1 - zb1)
        s4_buf[0, pl.ds(kb, S), :] = silu.astype(jnp.bfloat16)

        ex3 = start(s4_buf.at[0, pl.ds(kb, S), :],
                    s4_buf.at[0, pl.ds(kb, S), :],
                    ex_ss.at[2], ex_rs.at[2], d2)
        out_ref[pl.ds(out_base(own16), S), :] = silu
        ex3.wait_recv()
        ex4 = start(s4_buf.at[0, pl.ds(kh, R2), :],
                    s4_buf.at[0, pl.ds(kh, R2), :],
                    ex_ss.at[3], ex_rs.at[3], d1)
        out_ref[pl.ds(out_base(p16), S), :] = (
            s4_buf[0, pl.ds(sb, S), :].astype(jnp.float32)
        )
        ex4.wait_recv()
        sent4 = [(start(s4_buf.at[0, pl.ds(0, R2), :],
                        s4_buf.at[1, pl.ds(0, R2), :],
                        s4r_ss.at[0], s4r_rs.at[0], p_r),
                  start(s4_buf.at[0, pl.ds(R2, R2), :],
                        s4_buf.at[1, pl.ds(R2, R2), :],
                        s4l_ss.at[0], s4l_rs.at[0], p_l))]
        ohb = jnp.where(oh == 0, qR * Q, qL * Q + R2)
        out_ref[pl.ds(ohb, R2), :] = (
            s4_buf[0, pl.ds(oh, R2), :].astype(jnp.float32)
        )

        for s in range(3):
            rr, rl = sent4[s]
            rr.wait_recv()
            rl.wait_recv()
            if s < 2:
                sent4.append((
                    start(s4_buf.at[s + 1, pl.ds(0, R2), :],
                          s4_buf.at[s + 2, pl.ds(0, R2), :],
                          s4r_ss.at[s + 1], s4r_rs.at[s + 1], p_r),
                    start(s4_buf.at[s + 1, pl.ds(R2, R2), :],
                          s4_buf.at[s + 2, pl.ds(R2, R2), :],
                          s4l_ss.at[s + 1], s4l_rs.at[s + 1], p_l),
                ))
            rq = lax.rem(p - s + 4, 4)
            lq = lax.rem(p + s, 4)
            out_ref[pl.ds(rq * Q, R2), :] = (
                s4_buf[s + 1, pl.ds(0, R2), :].astype(jnp.float32)
            )
            out_ref[pl.ds(lq * Q + R2, R2), :] = (
                s4_buf[s + 1, pl.ds(R2, R2), :].astype(jnp.float32)
            )

        for r in pending:
            r.wait_send()

    return pl.pallas_call(
        body,
        out_shape=jax.ShapeDtypeStruct((M, N), jnp.float32),
        in_specs=[
            pl.BlockSpec(memory_space=pltpu.VMEM),
            pl.BlockSpec(memory_space=pltpu.VMEM),
        ],
        out_specs=pl.BlockSpec(memory_space=pltpu.VMEM),
        scratch_shapes=[
            pltpu.VMEM((4, Q, N), jnp.bfloat16),
            pltpu.VMEM((R2, N), jnp.bfloat16),
            pltpu.VMEM((S, N), jnp.bfloat16),
            pltpu.VMEM((4, Q, N), jnp.bfloat16),
            pltpu.SemaphoreType.DMA((3,)),
            pltpu.SemaphoreType.DMA((3,)),
            pltpu.SemaphoreType.DMA((3,)),
            pltpu.SemaphoreType.DMA((3,)),
            pltpu.SemaphoreType.DMA((4,)),
            pltpu.SemaphoreType.DMA((4,)),
            pltpu.SemaphoreType.DMA((3,)),
            pltpu.SemaphoreType.DMA((3,)),
            pltpu.SemaphoreType.DMA((3,)),
            pltpu.SemaphoreType.DMA((3,)),
        ],
        compiler_params=pltpu.CompilerParams(collective_id=0),
    )(x, w_mat)


# device time: 51402 ns/iter; 1.0540x vs baseline; 1.0540x over previous
import os

import jax
import jax.numpy as jnp
from jax import lax
from jax.experimental import pallas as pl
from jax.experimental.pallas import tpu as pltpu

_PROBE = os.environ.get("KERNEL_PROBE", "")

N_DEV = 16
M = 1024
N = 1024
Q = M // 4
S = M // 16
R2 = Q // 2


def kernel(x, w_mat):
    def body(x_ref, w_ref, out_ref, s1_buf, e1_buf, e2_buf, s4_buf,
             s1r_ss, s1r_rs, s1l_ss, s1l_rs,
             ex_ss, ex_rs,
             s4r_ss, s4r_rs, s4l_ss, s4l_rs):
        my = lax.axis_index("i")
        z = my // 4
        p = lax.rem(my, 4)
        zb0 = lax.rem(z, 2)
        zb1 = z // 2
        p_r = z * 4 + lax.rem(p + 1, 4)
        p_l = z * 4 + lax.rem(p + 3, 4)
        d1 = (z ^ 1) * 4 + p
        d2 = (z ^ 2) * 4 + p

        bs = pltpu.get_barrier_semaphore()
        for nbr in (p_l, p_r, d1, d2):
            pl.semaphore_signal(
                bs, inc=1, device_id=(nbr,),
                device_id_type=pl.DeviceIdType.MESH,
            )
        pl.semaphore_wait(bs, 4)

        pending = []

        def start(src, dst, ssem, rsem, dev):
            r = pltpu.make_async_remote_copy(
                src_ref=src, dst_ref=dst, send_sem=ssem, recv_sem=rsem,
                device_id=(dev,), device_id_type=pl.DeviceIdType.MESH,
            )
            r.start()
            pending.append(r)
            return r

        wv = w_ref[...]

        def gemm(row0, nrows):
            return jnp.dot(
                x_ref[pl.ds(row0, nrows), :], wv,
                preferred_element_type=jnp.float32,
            )

        SEGS = (0, S, R2, R2 + S)

        def seg_sems(gi, ss_r, ss_l, rs_r, rs_l):
            return ((ss_r, ss_l)[gi // 2].at[gi % 2],
                    (rs_r, rs_l)[gi // 2].at[gi % 2])

        s1_buf[0, :, :] = gemm(p * Q, Q).astype(jnp.bfloat16)
        if "noplane" not in _PROBE:
            cur = []
            for gi, off in enumerate(SEGS):
                ssem, rsem = seg_sems(gi, s1r_ss, s1l_ss, s1r_rs, s1l_rs)
                cur.append(start(s1_buf.at[0, pl.ds(off, S), :],
                                 s1_buf.at[1, pl.ds(off, S), :],
                                 ssem.at[0], rsem.at[0],
                                 p_r if gi < 2 else p_l))
            for s in range(3):
                nxt = []
                for gi, off in enumerate(SEGS):
                    ssem, rsem = seg_sems(gi, s1r_ss, s1l_ss, s1r_rs, s1l_rs)
                    qidx = (lax.rem(p + 3 - s, 4) if gi < 2
                            else lax.rem(p + 1 + s, 4))
                    contrib = gemm(qidx * Q + off, S)
                    cur[gi].wait_recv()
                    s1_buf[s + 1, pl.ds(off, S), :] = (
                        s1_buf[s + 1, pl.ds(off, S), :].astype(jnp.float32)
                        + contrib
                    ).astype(jnp.bfloat16)
                    if s < 2:
                        nxt.append(start(s1_buf.at[s + 1, pl.ds(off, S), :],
                                         s1_buf.at[s + 2, pl.ds(off, S), :],
                                         ssem.at[s + 1], rsem.at[s + 1],
                                         p_r if gi < 2 else p_l))
                cur = nxt
        qR = lax.rem(p + 1, 4)
        qL = lax.rem(p + 3, 4)

        def out_base(j):
            return jnp.where(j < 2, qR, qL) * Q + j * S

        _noz = "noz" in _PROBE
        kh = zb0 * R2
        oh = (1 - zb0) * R2
        if not _noz:
            ex1 = start(s1_buf.at[3, pl.ds(oh, R2), :], e1_buf,
                        ex_ss.at[0], ex_rs.at[0], d1)
            ex1.wait_recv()
            s1_buf[3, pl.ds(kh, R2), :] = (
                s1_buf[3, pl.ds(kh, R2), :].astype(jnp.float32)
                + e1_buf[...].astype(jnp.float32)
            ).astype(jnp.bfloat16)

        kb = kh + zb1 * S
        sb = kh + (1 - zb1) * S
        if not _noz:
            ex2 = start(s1_buf.at[3, pl.ds(sb, S), :], e2_buf,
                        ex_ss.at[1], ex_rs.at[1], d2)
            ex2.wait_recv()
        red = (s1_buf[3, pl.ds(kb, S), :].astype(jnp.float32)
               + e2_buf[...].astype(jnp.float32))
        silu = red * jax.nn.sigmoid(red)
        own16 = zb0 * 2 + zb1
        p16 = zb0 * 2 + (1 - zb1)
        s4_buf[0, pl.ds(kb, S), :] = silu.astype(jnp.bfloat16)

        if not _noz:
            ex3 = start(s4_buf.at[0, pl.ds(kb, S), :],
                        s4_buf.at[0, pl.ds(kb, S), :],
                        ex_ss.at[2], ex_rs.at[2], d2)
        out_ref[pl.ds(out_base(own16), S), :] = silu
        if not _noz:
            ex3.wait_recv()
            ex4 = start(s4_buf.at[0, pl.ds(kh, R2), :],
                        s4_buf.at[0, pl.ds(kh, R2), :],
                        ex_ss.at[3], ex_rs.at[3], d1)
        out_ref[pl.ds(out_base(p16), S), :] = (
            s4_buf[0, pl.ds(sb, S), :].astype(jnp.float32)
        )
        if not _noz:
            ex4.wait_recv()
        if "nos4" not in _PROBE:
            cur4 = []
            for gi, off in enumerate(SEGS):
                ssem, rsem = seg_sems(gi, s4r_ss, s4l_ss, s4r_rs, s4l_rs)
                cur4.append(start(s4_buf.at[0, pl.ds(off, S), :],
                                  s4_buf.at[1, pl.ds(off, S), :],
                                  ssem.at[0], rsem.at[0],
                                  p_r if gi < 2 else p_l))
        ohb = jnp.where(oh == 0, qR * Q, qL * Q + R2)
        out_ref[pl.ds(ohb, R2), :] = (
            s4_buf[0, pl.ds(oh, R2), :].astype(jnp.float32)
        )

        for s in range(3 if "nos4" not in _PROBE else 0):
            nxt4 = []
            for gi, off in enumerate(SEGS):
                ssem, rsem = seg_sems(gi, s4r_ss, s4l_ss, s4r_rs, s4l_rs)
                cur4[gi].wait_recv()
                if s < 2:
                    nxt4.append(start(s4_buf.at[s + 1, pl.ds(off, S), :],
                                      s4_buf.at[s + 2, pl.ds(off, S), :],
                                      ssem.at[s + 1], rsem.at[s + 1],
                                      p_r if gi < 2 else p_l))
                qidx = (lax.rem(p - s + 4, 4) if gi < 2
                        else lax.rem(p + s, 4))
                out_ref[pl.ds(qidx * Q + off, S), :] = (
                    s4_buf[s + 1, pl.ds(off, S), :].astype(jnp.float32)
                )
            cur4 = nxt4

        for r in pending:
            r.wait_send()

    return pl.pallas_call(
        body,
        out_shape=jax.ShapeDtypeStruct((M, N), jnp.float32),
        in_specs=[
            pl.BlockSpec(memory_space=pltpu.VMEM),
            pl.BlockSpec(memory_space=pltpu.VMEM),
        ],
        out_specs=pl.BlockSpec(memory_space=pltpu.VMEM),
        scratch_shapes=[
            pltpu.VMEM((4, Q, N), jnp.bfloat16),
            pltpu.VMEM((R2, N), jnp.bfloat16),
            pltpu.VMEM((S, N), jnp.bfloat16),
            pltpu.VMEM((4, Q, N), jnp.bfloat16),
            pltpu.SemaphoreType.DMA((2, 3)),
            pltpu.SemaphoreType.DMA((2, 3)),
            pltpu.SemaphoreType.DMA((2, 3)),
            pltpu.SemaphoreType.DMA((2, 3)),
            pltpu.SemaphoreType.DMA((4,)),
            pltpu.SemaphoreType.DMA((4,)),
            pltpu.SemaphoreType.DMA((2, 3)),
            pltpu.SemaphoreType.DMA((2, 3)),
            pltpu.SemaphoreType.DMA((2, 3)),
            pltpu.SemaphoreType.DMA((2, 3)),
        ],
        compiler_params=pltpu.CompilerParams(collective_id=0),
    )(x, w_mat)


# device time: 49929 ns/iter; 1.0851x vs baseline; 1.0295x over previous
import os

import jax
import jax.numpy as jnp
from jax import lax
from jax.experimental import pallas as pl
from jax.experimental.pallas import tpu as pltpu

_PROBE = os.environ.get("KERNEL_PROBE", "")

N_DEV = 16
M = 1024
N = 1024
Q = M // 4
S = M // 16
R2 = Q // 2


def kernel(x, w_mat):
    def body(x_ref, w_ref, out_ref, s1_buf, e1_buf, e2_buf, s4_buf,
             s1r_ss, s1r_rs, s1l_ss, s1l_rs,
             ex_ss, ex_rs,
             s4r_ss, s4r_rs, s4l_ss, s4l_rs):
        my = lax.axis_index("i")
        z = my // 4
        p = lax.rem(my, 4)
        zb0 = lax.rem(z, 2)
        zb1 = z // 2
        p_r = z * 4 + lax.rem(p + 1, 4)
        p_l = z * 4 + lax.rem(p + 3, 4)
        d1 = (z ^ 1) * 4 + p
        d2 = (z ^ 2) * 4 + p

        kh = zb0 * R2
        oh = (1 - zb0) * R2
        kb = kh + zb1 * S
        sb = kh + (1 - zb1) * S
        own16 = zb0 * 2 + zb1
        p16 = zb0 * 2 + (1 - zb1)
        qTOP = lax.rem(p + 1, 4)
        qBOT = lax.rem(p + 3, 4)
        dsh = 2 * zb0

        SEGS = (kh, kh + S, oh, oh + S)

        def seg_sems(gi, ss_r, ss_l, rs_r, rs_l):
            return ((ss_r, ss_l)[gi // 2].at[gi % 2],
                    (rs_r, rs_l)[gi // 2].at[gi % 2])

        def out_base(j):
            return jnp.where(j < 2, qTOP, qBOT) * Q + j * S

        pending = []

        def start(src, dst, ssem, rsem, dev):
            r = pltpu.make_async_remote_copy(
                src_ref=src, dst_ref=dst, send_sem=ssem, recv_sem=rsem,
                device_id=(dev,), device_id_type=pl.DeviceIdType.MESH,
            )
            r.start()
            pending.append(r)
            return r

        wv = w_ref[...]

        def gemm(row0, nrows):
            return jnp.dot(
                x_ref[pl.ds(row0, nrows), :], wv,
                preferred_element_type=jnp.float32,
            )

        s1_buf[0, :, :] = gemm(lax.rem(p + dsh, 4) * Q, Q).astype(jnp.bfloat16)

        bs = pltpu.get_barrier_semaphore()
        for nbr in (p_l, p_r, d1, d2):
            pl.semaphore_signal(
                bs, inc=1, device_id=(nbr,),
                device_id_type=pl.DeviceIdType.MESH,
            )
        pl.semaphore_wait(bs, 4)

        def s1_step_seg(s, gi, launch_next=True):
            off = SEGS[gi]
            ssem, rsem = seg_sems(gi, s1r_ss, s1l_ss, s1r_rs, s1l_rs)
            qidx = (lax.rem(p + 3 - s + dsh, 4) if gi < 2
                    else lax.rem(p + 1 + s + dsh, 4))
            contrib = gemm(qidx * Q + off, S)
            cur[gi].wait_recv()
            s1_buf[s + 1, pl.ds(off, S), :] = (
                s1_buf[s + 1, pl.ds(off, S), :].astype(jnp.float32)
                + contrib
            ).astype(jnp.bfloat16)
            if launch_next:
                nxt.append(start(s1_buf.at[s + 1, pl.ds(off, S), :],
                                 s1_buf.at[s + 2, pl.ds(off, S), :],
                                 ssem.at[s + 1], rsem.at[s + 1],
                                 p_r if gi < 2 else p_l))

        ex1 = None
        if "noplane" not in _PROBE:
            cur = []
            for gi, off in enumerate(SEGS):
                ssem, rsem = seg_sems(gi, s1r_ss, s1l_ss, s1r_rs, s1l_rs)
                cur.append(start(s1_buf.at[0, pl.ds(off, S), :],
                                 s1_buf.at[1, pl.ds(off, S), :],
                                 ssem.at[0], rsem.at[0],
                                 p_r if gi < 2 else p_l))
            for s in range(2):
                nxt = []
                for gi in range(4):
                    s1_step_seg(s, gi)
                cur = nxt
            nxt = []
            s1_step_seg(2, 2, launch_next=False)
            s1_step_seg(2, 3, launch_next=False)
            if "noz" not in _PROBE:
                ex1 = start(s1_buf.at[3, pl.ds(oh, R2), :], e1_buf,
                            ex_ss.at[0], ex_rs.at[0], d1)
            s1_step_seg(2, 0, launch_next=False)
            s1_step_seg(2, 1, launch_next=False)
        elif "noz" not in _PROBE:
            ex1 = start(s1_buf.at[3, pl.ds(oh, R2), :], e1_buf,
                        ex_ss.at[0], ex_rs.at[0], d1)

        if ex1 is not None:
            ex1.wait_recv()
            s1_buf[3, pl.ds(kh, R2), :] = (
                s1_buf[3, pl.ds(kh, R2), :].astype(jnp.float32)
                + e1_buf[...].astype(jnp.float32)
            ).astype(jnp.bfloat16)
            ex2 = start(s1_buf.at[3, pl.ds(sb, S), :], e2_buf,
                        ex_ss.at[1], ex_rs.at[1], d2)
            ex2.wait_recv()
        red = (s1_buf[3, pl.ds(kb, S), :].astype(jnp.float32)
               + e2_buf[...].astype(jnp.float32))
        silu = red * jax.nn.sigmoid(red)
        s4_buf[0, pl.ds(kb, S), :] = silu.astype(jnp.bfloat16)

        if ex1 is not None:
            ex3 = start(s4_buf.at[0, pl.ds(kb, S), :],
                        s4_buf.at[0, pl.ds(kb, S), :],
                        ex_ss.at[2], ex_rs.at[2], d2)
        out_ref[pl.ds(out_base(own16), S), :] = silu
        if ex1 is not None:
            ex3.wait_recv()
        if "nos4" not in _PROBE:
            cur4 = []
            for gi in (0, 1):
                off = SEGS[gi]
                ssem, rsem = seg_sems(gi, s4r_ss, s4l_ss, s4r_rs, s4l_rs)
                cur4.append(start(s4_buf.at[0, pl.ds(off, S), :],
                                  s4_buf.at[1, pl.ds(off, S), :],
                                  ssem.at[0], rsem.at[0], p_r))
        if ex1 is not None:
            ex4 = start(s4_buf.at[0, pl.ds(kh, R2), :],
                        s4_buf.at[0, pl.ds(kh, R2), :],
                        ex_ss.at[3], ex_rs.at[3], d1)
        out_ref[pl.ds(out_base(p16), S), :] = (
            s4_buf[0, pl.ds(sb, S), :].astype(jnp.float32)
        )
        if ex1 is not None:
            ex4.wait_recv()
        if "nos4" not in _PROBE:
            for gi in (2, 3):
                off = SEGS[gi]
                ssem, rsem = seg_sems(gi, s4r_ss, s4l_ss, s4r_rs, s4l_rs)
                cur4.append(start(s4_buf.at[0, pl.ds(off, S), :],
                                  s4_buf.at[1, pl.ds(off, S), :],
                                  ssem.at[0], rsem.at[0], p_l))
        out_ref[pl.ds(jnp.where(zb0 == 1, qTOP, qBOT) * Q + oh, R2), :] = (
            s4_buf[0, pl.ds(oh, R2), :].astype(jnp.float32)
        )

        for s in range(3 if "nos4" not in _PROBE else 0):
            nxt4 = []
            for gi in range(4):
                off = SEGS[gi]
                ssem, rsem = seg_sems(gi, s4r_ss, s4l_ss, s4r_rs, s4l_rs)
                cur4[gi].wait_recv()
                if s < 2:
                    nxt4.append(start(s4_buf.at[s + 1, pl.ds(off, S), :],
                                      s4_buf.at[s + 2, pl.ds(off, S), :],
                                      ssem.at[s + 1], rsem.at[s + 1],
                                      p_r if gi < 2 else p_l))
                qidx = (lax.rem(p - s + 4 + dsh, 4) if gi < 2
                        else lax.rem(p + s + dsh, 4))
                out_ref[pl.ds(qidx * Q + off, S), :] = (
                    s4_buf[s + 1, pl.ds(off, S), :].astype(jnp.float32)
                )
            cur4 = nxt4

        for r in pending:
            r.wait_send()

    return pl.pallas_call(
        body,
        out_shape=jax.ShapeDtypeStruct((M, N), jnp.float32),
        in_specs=[
            pl.BlockSpec(memory_space=pltpu.VMEM),
            pl.BlockSpec(memory_space=pltpu.VMEM),
        ],
        out_specs=pl.BlockSpec(memory_space=pltpu.VMEM),
        scratch_shapes=[
            pltpu.VMEM((4, Q, N), jnp.bfloat16),
            pltpu.VMEM((R2, N), jnp.bfloat16),
            pltpu.VMEM((S, N), jnp.bfloat16),
            pltpu.VMEM((4, Q, N), jnp.bfloat16),
            pltpu.SemaphoreType.DMA((2, 3)),
            pltpu.SemaphoreType.DMA((2, 3)),
            pltpu.SemaphoreType.DMA((2, 3)),
            pltpu.SemaphoreType.DMA((2, 3)),
            pltpu.SemaphoreType.DMA((4,)),
            pltpu.SemaphoreType.DMA((4,)),
            pltpu.SemaphoreType.DMA((2, 3)),
            pltpu.SemaphoreType.DMA((2, 3)),
            pltpu.SemaphoreType.DMA((2, 3)),
            pltpu.SemaphoreType.DMA((2, 3)),
        ],
        compiler_params=pltpu.CompilerParams(collective_id=0),
    )(x, w_mat)


# device time: 48493 ns/iter; 1.1172x vs baseline; 1.0296x over previous
import os

import jax
import jax.numpy as jnp
from jax import lax
from jax.experimental import pallas as pl
from jax.experimental.pallas import tpu as pltpu

_PROBE = os.environ.get("KERNEL_PROBE", "")

N_DEV = 16
M = 1024
N = 1024
Q = M // 4
S = M // 16
R2 = Q // 2


def kernel(x, w_mat):
    def body(x_ref, w_ref, out_ref, s1_buf, e1_buf, e2_buf, s4_buf,
             s1r_ss, s1r_rs, s1l_ss, s1l_rs,
             ex_ss, ex_rs,
             s4r_ss, s4r_rs, s4l_ss, s4l_rs):
        my = lax.axis_index("i")
        z = my // 4
        p = lax.rem(my, 4)
        zb0 = lax.rem(z, 2)
        zb1 = z // 2
        p_r = z * 4 + lax.rem(p + 1, 4)
        p_l = z * 4 + lax.rem(p + 3, 4)
        d1 = (z ^ 1) * 4 + p
        d2 = (z ^ 2) * 4 + p

        kh = zb0 * R2
        oh = (1 - zb0) * R2
        kb = kh + zb1 * S
        sb = kh + (1 - zb1) * S
        own16 = zb0 * 2 + zb1
        p16 = zb0 * 2 + (1 - zb1)
        qTOP = lax.rem(p + 1, 4)
        qBOT = lax.rem(p + 3, 4)
        dsh = 2 * zb0

        SEGS = (kh, kh + S, oh, oh + S)

        def seg_sems(gi, ss_r, ss_l, rs_r, rs_l):
            return ((ss_r, ss_l)[gi // 2].at[gi % 2],
                    (rs_r, rs_l)[gi // 2].at[gi % 2])

        def out_base(j):
            return jnp.where(j < 2, qTOP, qBOT) * Q + j * S

        pending = []

        def start(src, dst, ssem, rsem, dev):
            r = pltpu.make_async_remote_copy(
                src_ref=src, dst_ref=dst, send_sem=ssem, recv_sem=rsem,
                device_id=(dev,), device_id_type=pl.DeviceIdType.MESH,
            )
            r.start()
            pending.append(r)
            return r

        wv = w_ref[...]

        def gemm(row0, nrows):
            return jnp.dot(
                x_ref[pl.ds(row0, nrows), :], wv,
                preferred_element_type=jnp.float32,
            )

        s1_buf[0, :, :] = gemm(lax.rem(p + dsh, 4) * Q, Q).astype(jnp.bfloat16)

        bs = pltpu.get_barrier_semaphore()
        for nbr in (p_l, p_r, d1, d2):
            pl.semaphore_signal(
                bs, inc=1, device_id=(nbr,),
                device_id_type=pl.DeviceIdType.MESH,
            )
        pl.semaphore_wait(bs, 4)

        def s1_step_seg(s, gi, launch_next=True):
            off = SEGS[gi]
            ssem, rsem = seg_sems(gi, s1r_ss, s1l_ss, s1r_rs, s1l_rs)
            qidx = (lax.rem(p + 3 - s + dsh, 4) if gi < 2
                    else lax.rem(p + 1 + s + dsh, 4))
            contrib = gemm(qidx * Q + off, S)
            cur[gi].wait_recv()
            s1_buf[s + 1, pl.ds(off, S), :] = (
                s1_buf[s + 1, pl.ds(off, S), :].astype(jnp.float32)
                + contrib
            ).astype(jnp.bfloat16)
            if launch_next:
                nxt.append(start(s1_buf.at[s + 1, pl.ds(off, S), :],
                                 s1_buf.at[s + 2, pl.ds(off, S), :],
                                 ssem.at[s + 1], rsem.at[s + 1],
                                 p_r if gi < 2 else p_l))

        ex1 = None
        if "noplane" not in _PROBE:
            cur = [None] * 4
            for gi in (2, 3, 0, 1):
                off = SEGS[gi]
                ssem, rsem = seg_sems(gi, s1r_ss, s1l_ss, s1r_rs, s1l_rs)
                cur[gi] = start(s1_buf.at[0, pl.ds(off, S), :],
                                s1_buf.at[1, pl.ds(off, S), :],
                                ssem.at[0], rsem.at[0],
                                p_r if gi < 2 else p_l)
            for s in range(2):
                nxt = []
                for gi in (2, 3, 0, 1):
                    s1_step_seg(s, gi)
                nxt = [nxt[2], nxt[3], nxt[0], nxt[1]]
                cur = nxt
            nxt = []
            s1_step_seg(2, 2, launch_next=False)
            s1_step_seg(2, 3, launch_next=False)
            if "noz" not in _PROBE:
                ex1 = start(s1_buf.at[3, pl.ds(oh, R2), :], e1_buf,
                            ex_ss.at[0], ex_rs.at[0], d1)
            s1_step_seg(2, 0, launch_next=False)
            s1_step_seg(2, 1, launch_next=False)
        elif "noz" not in _PROBE:
            ex1 = start(s1_buf.at[3, pl.ds(oh, R2), :], e1_buf,
                        ex_ss.at[0], ex_rs.at[0], d1)

        if ex1 is not None:
            ex1.wait_recv()
            s1_buf[3, pl.ds(kh, R2), :] = (
                s1_buf[3, pl.ds(kh, R2), :].astype(jnp.float32)
                + e1_buf[...].astype(jnp.float32)
            ).astype(jnp.bfloat16)
            ex2 = start(s1_buf.at[3, pl.ds(sb, S), :], e2_buf,
                        ex_ss.at[1], ex_rs.at[1], d2)
            ex2.wait_recv()
        red = (s1_buf[3, pl.ds(kb, S), :].astype(jnp.float32)
               + e2_buf[...].astype(jnp.float32))
        silu = red * jax.nn.sigmoid(red)
        s4_buf[0, pl.ds(kb, S), :] = silu.astype(jnp.bfloat16)

        if ex1 is not None:
            ex3 = start(s4_buf.at[0, pl.ds(kb, S), :],
                        s4_buf.at[0, pl.ds(kb, S), :],
                        ex_ss.at[2], ex_rs.at[2], d2)
        out_ref[pl.ds(out_base(own16), S), :] = silu
        if ex1 is not None:
            ex3.wait_recv()
        if "nos4" not in _PROBE:
            cur4 = []
            for gi in (0, 1):
                off = SEGS[gi]
                ssem, rsem = seg_sems(gi, s4r_ss, s4l_ss, s4r_rs, s4l_rs)
                cur4.append(start(s4_buf.at[0, pl.ds(off, S), :],
                                  s4_buf.at[1, pl.ds(off, S), :],
                                  ssem.at[0], rsem.at[0], p_r))
        if ex1 is not None:
            ex4a = start(s4_buf.at[0, pl.ds(kh, S), :],
                         s4_buf.at[0, pl.ds(kh, S), :],
                         ex_ss.at[3], ex_rs.at[3], d1)
            ex4b = start(s4_buf.at[0, pl.ds(kh + S, S), :],
                         s4_buf.at[0, pl.ds(kh + S, S), :],
                         ex_ss.at[4], ex_rs.at[4], d1)
        out_ref[pl.ds(out_base(p16), S), :] = (
            s4_buf[0, pl.ds(sb, S), :].astype(jnp.float32)
        )
        for gi, ex in ((2, "a"), (3, "b")):
            if ex1 is not None:
                (ex4a if ex == "a" else ex4b).wait_recv()
            if "nos4" not in _PROBE:
                off = SEGS[gi]
                ssem, rsem = seg_sems(gi, s4r_ss, s4l_ss, s4r_rs, s4l_rs)
                cur4.append(start(s4_buf.at[0, pl.ds(off, S), :],
                                  s4_buf.at[1, pl.ds(off, S), :],
                                  ssem.at[0], rsem.at[0], p_l))
        out_ref[pl.ds(jnp.where(zb0 == 1, qTOP, qBOT) * Q + oh, R2), :] = (
            s4_buf[0, pl.ds(oh, R2), :].astype(jnp.float32)
        )

        for s in range(3 if "nos4" not in _PROBE else 0):
            nxt4 = [None] * 4
            for gi in (2, 3, 0, 1):
                off = SEGS[gi]
                ssem, rsem = seg_sems(gi, s4r_ss, s4l_ss, s4r_rs, s4l_rs)
                cur4[gi].wait_recv()
                if s < 2:
                    nxt4[gi] = start(s4_buf.at[s + 1, pl.ds(off, S), :],
                                     s4_buf.at[s + 2, pl.ds(off, S), :],
                                     ssem.at[s + 1], rsem.at[s + 1],
                                     p_r if gi < 2 else p_l)
                qidx = (lax.rem(p - s + 4 + dsh, 4) if gi < 2
                        else lax.rem(p + s + dsh, 4))
                out_ref[pl.ds(qidx * Q + off, S), :] = (
                    s4_buf[s + 1, pl.ds(off, S), :].astype(jnp.float32)
                )
            cur4 = nxt4

        for r in pending:
            r.wait_send()

    return pl.pallas_call(
        body,
        out_shape=jax.ShapeDtypeStruct((M, N), jnp.float32),
        in_specs=[
            pl.BlockSpec(memory_space=pltpu.VMEM),
            pl.BlockSpec(memory_space=pltpu.VMEM),
        ],
        out_specs=pl.BlockSpec(memory_space=pltpu.VMEM),
        scratch_shapes=[
            pltpu.VMEM((4, Q, N), jnp.bfloat16),
            pltpu.VMEM((R2, N), jnp.bfloat16),
            pltpu.VMEM((S, N), jnp.bfloat16),
            pltpu.VMEM((4, Q, N), jnp.bfloat16),
            pltpu.SemaphoreType.DMA((2, 3)),
            pltpu.SemaphoreType.DMA((2, 3)),
            pltpu.SemaphoreType.DMA((2, 3)),
            pltpu.SemaphoreType.DMA((2, 3)),
            pltpu.SemaphoreType.DMA((5,)),
            pltpu.SemaphoreType.DMA((5,)),
            pltpu.SemaphoreType.DMA((2, 3)),
            pltpu.SemaphoreType.DMA((2, 3)),
            pltpu.SemaphoreType.DMA((2, 3)),
            pltpu.SemaphoreType.DMA((2, 3)),
        ],
        compiler_params=pltpu.CompilerParams(collective_id=0),
    )(x, w_mat)


# device time: 48061 ns/iter; 1.1272x vs baseline; 1.0090x over previous
import os

import jax
import jax.numpy as jnp
from jax import lax
from jax.experimental import pallas as pl
from jax.experimental.pallas import tpu as pltpu

_PROBE = os.environ.get("KERNEL_PROBE", "")

N_DEV = 16
M = 1024
N = 1024
Q = M // 4
S = M // 16
R2 = Q // 2


def kernel(x, w_mat):
    def body(x_ref, w_ref, out_ref, s1_buf, e1_buf, e2_buf, s4_buf,
             s1r_ss, s1r_rs, s1l_ss, s1l_rs,
             ex_ss, ex_rs,
             s4r_ss, s4r_rs, s4l_ss, s4l_rs):
        my = lax.axis_index("i")
        z = my // 4
        p = lax.rem(my, 4)
        zb0 = lax.rem(z, 2)
        zb1 = z // 2
        p_r = z * 4 + lax.rem(p + 1, 4)
        p_l = z * 4 + lax.rem(p + 3, 4)
        d1 = (z ^ 1) * 4 + p
        d2 = (z ^ 2) * 4 + p

        kh = zb0 * R2
        oh = (1 - zb0) * R2
        kb = kh + zb1 * S
        sb = kh + (1 - zb1) * S
        own16 = zb0 * 2 + zb1
        p16 = zb0 * 2 + (1 - zb1)
        qTOP = lax.rem(p + 1, 4)
        qBOT = lax.rem(p + 3, 4)
        dsh = 2 * zb0

        SEGS = (kh, kh + S, oh, oh + S)

        def seg_sems(gi, ss_r, ss_l, rs_r, rs_l):
            return ((ss_r, ss_l)[gi // 2].at[gi % 2],
                    (rs_r, rs_l)[gi // 2].at[gi % 2])

        def out_base(j):
            return jnp.where(j < 2, qTOP, qBOT) * Q + j * S

        pending = []

        def start(src, dst, ssem, rsem, dev):
            r = pltpu.make_async_remote_copy(
                src_ref=src, dst_ref=dst, send_sem=ssem, recv_sem=rsem,
                device_id=(dev,), device_id_type=pl.DeviceIdType.MESH,
            )
            r.start()
            pending.append(r)
            return r

        wv = w_ref[...]

        def gemm(row0, nrows):
            return jnp.dot(
                x_ref[pl.ds(row0, nrows), :], wv,
                preferred_element_type=jnp.float32,
            )

        bs = pltpu.get_barrier_semaphore()
        for nbr in (p_l, p_r, d1, d2):
            pl.semaphore_signal(
                bs, inc=1, device_id=(nbr,),
                device_id_type=pl.DeviceIdType.MESH,
            )
        s1_buf[0, :, :] = gemm(lax.rem(p + dsh, 4) * Q, Q).astype(jnp.bfloat16)
        pl.semaphore_wait(bs, 4)

        def s1_step_seg(s, gi, launch_next=True):
            off = SEGS[gi]
            ssem, rsem = seg_sems(gi, s1r_ss, s1l_ss, s1r_rs, s1l_rs)
            qidx = (lax.rem(p + 3 - s + dsh, 4) if gi < 2
                    else lax.rem(p + 1 + s + dsh, 4))
            contrib = gemm(qidx * Q + off, S).astype(jnp.bfloat16)
            cur[gi].wait_recv()
            s1_buf[s + 1, pl.ds(off, S), :] = (
                s1_buf[s + 1, pl.ds(off, S), :] + contrib
            )
            if launch_next:
                nxt.append(start(s1_buf.at[s + 1, pl.ds(off, S), :],
                                 s1_buf.at[s + 2, pl.ds(off, S), :],
                                 ssem.at[s + 1], rsem.at[s + 1],
                                 p_r if gi < 2 else p_l))

        ex1 = None
        if "noplane" not in _PROBE:
            cur = [None] * 4
            for gi in (2, 3, 0, 1):
                off = SEGS[gi]
                ssem, rsem = seg_sems(gi, s1r_ss, s1l_ss, s1r_rs, s1l_rs)
                cur[gi] = start(s1_buf.at[0, pl.ds(off, S), :],
                                s1_buf.at[1, pl.ds(off, S), :],
                                ssem.at[0], rsem.at[0],
                                p_r if gi < 2 else p_l)
            for s in range(2):
                nxt = []
                for gi in (2, 3, 0, 1):
                    s1_step_seg(s, gi)
                nxt = [nxt[2], nxt[3], nxt[0], nxt[1]]
                cur = nxt
            nxt = []
            s1_step_seg(2, 2, launch_next=False)
            s1_step_seg(2, 3, launch_next=False)
            if "noz" not in _PROBE:
                ex1 = start(s1_buf.at[3, pl.ds(oh, R2), :], e1_buf,
                            ex_ss.at[0], ex_rs.at[0], d1)
            s1_step_seg(2, 0, launch_next=False)
            s1_step_seg(2, 1, launch_next=False)
        elif "noz" not in _PROBE:
            ex1 = start(s1_buf.at[3, pl.ds(oh, R2), :], e1_buf,
                        ex_ss.at[0], ex_rs.at[0], d1)

        if ex1 is not None:
            ex1.wait_recv()
            s1_buf[3, pl.ds(kh, R2), :] = (
                s1_buf[3, pl.ds(kh, R2), :] + e1_buf[...]
            )
            ex2 = start(s1_buf.at[3, pl.ds(sb, S), :], e2_buf,
                        ex_ss.at[1], ex_rs.at[1], d2)
            ex2.wait_recv()
        red = (s1_buf[3, pl.ds(kb, S), :].astype(jnp.float32)
               + e2_buf[...].astype(jnp.float32))
        silu = red * jax.nn.sigmoid(red)
        s4_buf[0, pl.ds(kb, S), :] = silu.astype(jnp.bfloat16)

        if ex1 is not None:
            ex3 = start(s4_buf.at[0, pl.ds(kb, S), :],
                        s4_buf.at[0, pl.ds(kb, S), :],
                        ex_ss.at[2], ex_rs.at[2], d2)
        out_ref[pl.ds(out_base(own16), S), :] = silu
        if ex1 is not None:
            ex3.wait_recv()
        if "nos4" not in _PROBE:
            cur4 = []
            for gi in (0, 1):
                off = SEGS[gi]
                ssem, rsem = seg_sems(gi, s4r_ss, s4l_ss, s4r_rs, s4l_rs)
                cur4.append(start(s4_buf.at[0, pl.ds(off, S), :],
                                  s4_buf.at[1, pl.ds(off, S), :],
                                  ssem.at[0], rsem.at[0], p_r))
        if ex1 is not None:
            ex4a = start(s4_buf.at[0, pl.ds(kh, S), :],
                         s4_buf.at[0, pl.ds(kh, S), :],
                         ex_ss.at[3], ex_rs.at[3], d1)
            ex4b = start(s4_buf.at[0, pl.ds(kh + S, S), :],
                         s4_buf.at[0, pl.ds(kh + S, S), :],
                         ex_ss.at[4], ex_rs.at[4], d1)
        out_ref[pl.ds(out_base(p16), S), :] = (
            s4_buf[0, pl.ds(sb, S), :].astype(jnp.float32)
        )
        for gi, ex in ((2, "a"), (3, "b")):
            if ex1 is not None:
                (ex4a if ex == "a" else ex4b).wait_recv()
            if "nos4" not in _PROBE:
                off = SEGS[gi]
                ssem, rsem = seg_sems(gi, s4r_ss, s4l_ss, s4r_rs, s4l_rs)
                cur4.append(start(s4_buf.at[0, pl.ds(off, S), :],
                                  s4_buf.at[1, pl.ds(off, S), :],
                                  ssem.at[0], rsem.at[0], p_l))
        out_ref[pl.ds(jnp.where(zb0 == 1, qTOP, qBOT) * Q + oh, R2), :] = (
            s4_buf[0, pl.ds(oh, R2), :].astype(jnp.float32)
        )

        for s in range(3 if "nos4" not in _PROBE else 0):
            nxt4 = [None] * 4
            for gi in (2, 3, 0, 1):
                off = SEGS[gi]
                ssem, rsem = seg_sems(gi, s4r_ss, s4l_ss, s4r_rs, s4l_rs)
                cur4[gi].wait_recv()
                if s < 2:
                    nxt4[gi] = start(s4_buf.at[s + 1, pl.ds(off, S), :],
                                     s4_buf.at[s + 2, pl.ds(off, S), :],
                                     ssem.at[s + 1], rsem.at[s + 1],
                                     p_r if gi < 2 else p_l)
                qidx = (lax.rem(p - s + 4 + dsh, 4) if gi < 2
                        else lax.rem(p + s + dsh, 4))
                out_ref[pl.ds(qidx * Q + off, S), :] = (
                    s4_buf[s + 1, pl.ds(off, S), :].astype(jnp.float32)
                )
            cur4 = nxt4

        for r in pending:
            r.wait_send()

    return pl.pallas_call(
        body,
        out_shape=jax.ShapeDtypeStruct((M, N), jnp.float32),
        in_specs=[
            pl.BlockSpec(memory_space=pltpu.VMEM),
            pl.BlockSpec(memory_space=pltpu.VMEM),
        ],
        out_specs=pl.BlockSpec(memory_space=pltpu.VMEM),
        scratch_shapes=[
            pltpu.VMEM((4, Q, N), jnp.bfloat16),
            pltpu.VMEM((R2, N), jnp.bfloat16),
            pltpu.VMEM((S, N), jnp.bfloat16),
            pltpu.VMEM((4, Q, N), jnp.bfloat16),
            pltpu.SemaphoreType.DMA((2, 3)),
            pltpu.SemaphoreType.DMA((2, 3)),
            pltpu.SemaphoreType.DMA((2, 3)),
            pltpu.SemaphoreType.DMA((2, 3)),
            pltpu.SemaphoreType.DMA((5,)),
            pltpu.SemaphoreType.DMA((5,)),
            pltpu.SemaphoreType.DMA((2, 3)),
            pltpu.SemaphoreType.DMA((2, 3)),
            pltpu.SemaphoreType.DMA((2, 3)),
            pltpu.SemaphoreType.DMA((2, 3)),
        ],
        compiler_params=pltpu.CompilerParams(collective_id=0),
    )(x, w_mat)
